# baseline (speedup 1.0000x reference)
import jax
import jax.numpy as jnp
from jax.experimental import pallas as pl
from jax.experimental.pallas import tpu as pltpu

_B = 8
_T = 4
_P = 1025
_H = 1280
_AR = 9
_PH = _P * _H
_ROWS = _T * _PH // 128
_RB = 1640
_NB = _ROWS // _RB


def _body(ids_ref, h_ref, t_ref, e_ref, gate_ref, o_ref):
    g = jnp.tanh(gate_ref[0])
    o_ref[...] = h_ref[...] + (1.0 - g) * e_ref[...] + g * t_ref[...]


def kernel(hidden_state, aspect_ratio_ids, gate, embedding, tile_embedding_weight):
    hv = hidden_state.reshape(_B, _ROWS, 128)
    tv = tile_embedding_weight.reshape(_AR, _ROWS, 128)
    ev = jnp.tile(embedding.reshape(-1), _T).reshape(_ROWS, 128)
    ids = aspect_ratio_ids.astype(jnp.int32)

    grid_spec = pltpu.PrefetchScalarGridSpec(
        num_scalar_prefetch=1,
        grid=(_B, _NB),
        in_specs=[
            pl.BlockSpec((1, _RB, 128), lambda b, j, ids: (b, j, 0)),
            pl.BlockSpec((1, _RB, 128), lambda b, j, ids: (ids[b], j, 0)),
            pl.BlockSpec((_RB, 128), lambda b, j, ids: (j, 0)),
            pl.BlockSpec(memory_space=pltpu.SMEM),
        ],
        out_specs=pl.BlockSpec((1, _RB, 128), lambda b, j, ids: (b, j, 0)),
    )
    out = pl.pallas_call(
        _body,
        grid_spec=grid_spec,
        out_shape=jax.ShapeDtypeStruct((_B, _ROWS, 128), jnp.float32),
        compiler_params=pltpu.CompilerParams(
            dimension_semantics=("arbitrary", "arbitrary"),
        ),
    )(ids, hv, tv, ev, gate)
    return out.reshape(_B, _T, _P, _H)


# speedup vs baseline: 2.0050x; 2.0050x over previous
import jax
import jax.numpy as jnp
from jax import lax
from jax.experimental import pallas as pl
from jax.experimental.pallas import tpu as pltpu
from jax.experimental.pallas import tpu_sc as plsc

_B = 8
_T = 4
_P = 1025
_H = 1280
_AR = 9
_PH = _P * _H
_PR = 16


def _sc_body(hid, ids, gate16, emb, table, out, embv, hv, tv, gv, idv,
             hsem, tsem, osem):
    c = lax.axis_index("c")
    s = lax.axis_index("s")
    wid = s * 2 + c

    pltpu.sync_copy(ids, idv)
    pltpu.sync_copy(gate16, gv)
    g = 1.0 - 2.0 / (jnp.exp(2.0 * gv[...]) + 1.0)
    one_m_g = 1.0 - g

    def do_slice(si, rows):
        cw = rows * _H
        nvec = cw // 16
        hpr = _H // 16
        p0 = si * _PR
        pltpu.sync_copy(emb.at[pl.ds(p0, rows), :], embv.at[pl.ds(0, rows), :])

        def slab(i, _):
            b = i // _T
            t = i % _T
            cp_h = pltpu.make_async_copy(
                hid.at[b, t, pl.ds(p0, rows), :], hv.at[pl.ds(0, rows), :], hsem)
            cp_h.start()
            cp_t = pltpu.make_async_copy(
                table.at[idv.at[pl.ds(b * 8, 1)], pl.ds(t * _PH + p0 * _H, cw)],
                tv.at[:, pl.ds(0, cw)], tsem)
            cp_t.start()
            cp_h.wait()
            cp_t.wait()

            def vec(k, _):
                row = k // hpr
                col = (k % hpr) * 16
                res = (hv[row, pl.ds(col, 16)]
                       + one_m_g * embv[row, pl.ds(col, 16)]
                       + g * tv[0, pl.ds(k * 16, 16)])
                hv[row, pl.ds(col, 16)] = res
                return 0

            lax.fori_loop(0, nvec, vec, 0)
            cp_o = pltpu.make_async_copy(
                hv.at[pl.ds(0, rows), :], out.at[b, t, pl.ds(p0, rows), :], osem)
            cp_o.start()
            cp_o.wait()
            return 0

        lax.fori_loop(0, _B * _T, slab, 0)

    do_slice(wid, _PR)
    do_slice(wid + 32, _PR)

    @pl.when(wid == 0)
    def _():
        do_slice(64, 1)


def kernel(hidden_state, aspect_ratio_ids, gate, embedding, tile_embedding_weight):
    ids = jnp.zeros((_B * 8,), jnp.int32).at[::8].set(
        aspect_ratio_ids.astype(jnp.int32))
    gate16 = jnp.broadcast_to(gate, (16,))

    mesh = plsc.VectorSubcoreMesh(core_axis_name="c", subcore_axis_name="s")
    sc = pl.kernel(
        _sc_body,
        out_type=jax.ShapeDtypeStruct((_B, _T, _P, _H), jnp.float32),
        mesh=mesh,
        scratch_types=[
            pltpu.VMEM((_PR, _H), jnp.float32),
            pltpu.VMEM((_PR, _H), jnp.float32),
            pltpu.VMEM((1, _PR * _H), jnp.float32),
            pltpu.VMEM((16,), jnp.float32),
            pltpu.VMEM((_B * 8,), jnp.int32),
            pltpu.SemaphoreType.DMA,
            pltpu.SemaphoreType.DMA,
            pltpu.SemaphoreType.DMA,
        ],
        compiler_params=pltpu.CompilerParams(use_tc_tiling_on_sc=True),
    )
    return sc(hidden_state, ids, gate16, embedding, tile_embedding_weight)


# speedup vs baseline: 2.2999x; 1.1471x over previous
import jax
import jax.numpy as jnp
from jax import lax
from jax.experimental import pallas as pl
from jax.experimental.pallas import tpu as pltpu
from jax.experimental.pallas import tpu_sc as plsc

_B = 8
_T = 4
_P = 1025
_H = 1280
_PH = _P * _H
_PR = 16
_HPR = _H // 16


def _sc_body(hid, ids, gate16, emb, table, out,
             embv, hva, hvb, tva, tvb, gv, idv,
             hsa, tsa, osa, hsb, tsb, osb):
    c = lax.axis_index("c")
    s = lax.axis_index("s")
    wid = s * 2 + c

    pltpu.sync_copy(ids, idv)
    pltpu.sync_copy(gate16, gv)
    g = 1.0 - 2.0 / (jnp.exp(2.0 * gv[...]) + 1.0)
    one_m_g = 1.0 - g

    def start_in(i, hv, tv, hsem, tsem, p0, rows, cw):
        b = i // _T
        t = i % _T
        pltpu.make_async_copy(
            hid.at[b, t, pl.ds(p0, rows), :], hv.at[pl.ds(0, rows), :], hsem
        ).start()
        pltpu.make_async_copy(
            table.at[idv.at[pl.ds(b * 8, 1)], pl.ds(t * _PH + p0 * _H, cw)],
            tv.at[:, pl.ds(0, cw)], tsem).start()

    def wait_in(i, hv, tv, hsem, tsem, p0, rows, cw):
        b = i // _T
        t = i % _T
        pltpu.make_async_copy(
            hid.at[b, t, pl.ds(p0, rows), :], hv.at[pl.ds(0, rows), :], hsem
        ).wait()
        pltpu.make_async_copy(
            table.at[idv.at[pl.ds(b * 8, 1)], pl.ds(t * _PH + p0 * _H, cw)],
            tv.at[:, pl.ds(0, cw)], tsem).wait()

    def compute(hv, tv, nvec):
        def vec(k, _):
            row = k // _HPR
            col = (k % _HPR) * 16
            hv[row, pl.ds(col, 16)] = (
                hv[row, pl.ds(col, 16)]
                + embv[row, pl.ds(col, 16)]
                + g * tv[0, pl.ds(k * 16, 16)])
            return 0
        lax.fori_loop(0, nvec, vec, 0)

    def out_copy(i, hv, osem, p0, rows):
        b = i // _T
        t = i % _T
        return pltpu.make_async_copy(
            hv.at[pl.ds(0, rows), :], out.at[b, t, pl.ds(p0, rows), :], osem)

    def do_slice(si, rows):
        cw = rows * _H
        nvec = cw // 16
        p0 = si * _PR
        pltpu.sync_copy(emb.at[pl.ds(p0, rows), :], embv.at[pl.ds(0, rows), :])

        def escale(k, _):
            row = k // _HPR
            col = (k % _HPR) * 16
            embv[row, pl.ds(col, 16)] = one_m_g * embv[row, pl.ds(col, 16)]
            return 0
        lax.fori_loop(0, nvec, escale, 0)

        start_in(0, hva, tva, hsa, tsa, p0, rows, cw)

        def pair(j, _):
            i0 = 2 * j
            i1 = 2 * j + 1

            @pl.when(j > 0)
            def _():
                out_copy(i1 - 2, hvb, osb, p0, rows).wait()
            start_in(i1, hvb, tvb, hsb, tsb, p0, rows, cw)

            wait_in(i0, hva, tva, hsa, tsa, p0, rows, cw)
            compute(hva, tva, nvec)
            out_copy(i0, hva, osa, p0, rows).start()

            @pl.when(j < _B * _T // 2 - 1)
            def _():
                out_copy(i0, hva, osa, p0, rows).wait()
                start_in(i0 + 2, hva, tva, hsa, tsa, p0, rows, cw)

            wait_in(i1, hvb, tvb, hsb, tsb, p0, rows, cw)
            compute(hvb, tvb, nvec)
            out_copy(i1, hvb, osb, p0, rows).start()
            return 0

        lax.fori_loop(0, _B * _T // 2, pair, 0)
        out_copy(_B * _T - 2, hva, osa, p0, rows).wait()
        out_copy(_B * _T - 1, hvb, osb, p0, rows).wait()

    do_slice(wid, _PR)
    do_slice(wid + 32, _PR)

    @pl.when(wid == 0)
    def _():
        do_slice(64, 1)


def kernel(hidden_state, aspect_ratio_ids, gate, embedding, tile_embedding_weight):
    ids = jnp.zeros((_B * 8,), jnp.int32).at[::8].set(
        aspect_ratio_ids.astype(jnp.int32))
    gate16 = jnp.broadcast_to(gate, (16,))

    mesh = plsc.VectorSubcoreMesh(core_axis_name="c", subcore_axis_name="s")
    sc = pl.kernel(
        _sc_body,
        out_type=jax.ShapeDtypeStruct((_B, _T, _P, _H), jnp.float32),
        mesh=mesh,
        scratch_types=[
            pltpu.VMEM((_PR, _H), jnp.float32),
            pltpu.VMEM((_PR, _H), jnp.float32),
            pltpu.VMEM((_PR, _H), jnp.float32),
            pltpu.VMEM((1, _PR * _H), jnp.float32),
            pltpu.VMEM((1, _PR * _H), jnp.float32),
            pltpu.VMEM((16,), jnp.float32),
            pltpu.VMEM((_B * 8,), jnp.int32),
            pltpu.SemaphoreType.DMA,
            pltpu.SemaphoreType.DMA,
            pltpu.SemaphoreType.DMA,
            pltpu.SemaphoreType.DMA,
            pltpu.SemaphoreType.DMA,
            pltpu.SemaphoreType.DMA,
        ],
        compiler_params=pltpu.CompilerParams(use_tc_tiling_on_sc=True),
    )
    return sc(hidden_state, ids, gate16, embedding, tile_embedding_weight)


# speedup vs baseline: 3.8049x; 1.6544x over previous
import jax
import jax.numpy as jnp
from jax import lax
from jax.experimental import pallas as pl
from jax.experimental.pallas import tpu as pltpu
from jax.experimental.pallas import tpu_sc as plsc

_B = 8
_T = 4
_P = 1025
_H = 1280
_PH = _P * _H
_PR = 16
_HPR = _H // 16


def _sc_body(hid, ids, gate16, emb, table, out,
             embv, hva, hvb, tva, tvb, gv, idv,
             hsa, tsa, osa, hsb, tsb, osb):
    c = lax.axis_index("c")
    s = lax.axis_index("s")
    wid = s * 2 + c

    pltpu.sync_copy(ids, idv)
    pltpu.sync_copy(gate16, gv)
    g = 1.0 - 2.0 / (jnp.exp(2.0 * gv[...]) + 1.0)
    one_m_g = 1.0 - g

    def start_in(i, hv, tv, hsem, tsem, p0, rows, cw):
        b = i // _T
        t = i % _T
        pltpu.make_async_copy(
            hid.at[b, t, pl.ds(p0, rows), :], hv.at[pl.ds(0, rows), :], hsem
        ).start()
        pltpu.make_async_copy(
            table.at[idv.at[pl.ds(b * 8, 1)], pl.ds(t * _PH + p0 * _H, cw)],
            tv.at[:, pl.ds(0, cw)], tsem).start()

    def wait_in(i, hv, tv, hsem, tsem, p0, rows, cw):
        b = i // _T
        t = i % _T
        pltpu.make_async_copy(
            hid.at[b, t, pl.ds(p0, rows), :], hv.at[pl.ds(0, rows), :], hsem
        ).wait()
        pltpu.make_async_copy(
            table.at[idv.at[pl.ds(b * 8, 1)], pl.ds(t * _PH + p0 * _H, cw)],
            tv.at[:, pl.ds(0, cw)], tsem).wait()

    def compute(hv, tv, rows):
        def row_body(row, _):
            base = row * _H

            @plsc.parallel_loop(0, _H, step=16, unroll=8)
            def _(col):
                hv[row, pl.ds(col, 16)] = (
                    hv[row, pl.ds(col, 16)]
                    + embv[row, pl.ds(col, 16)]
                    + g * tv[0, pl.ds(base + col, 16)])

            return 0
        lax.fori_loop(0, rows, row_body, 0)

    def out_copy(i, hv, osem, p0, rows):
        b = i // _T
        t = i % _T
        return pltpu.make_async_copy(
            hv.at[pl.ds(0, rows), :], out.at[b, t, pl.ds(p0, rows), :], osem)

    def do_slice(si, rows):
        cw = rows * _H
        nvec = cw // 16
        p0 = si * _PR
        pltpu.sync_copy(emb.at[pl.ds(p0, rows), :], embv.at[pl.ds(0, rows), :])

        def escale(row, _):
            @plsc.parallel_loop(0, _H, step=16, unroll=8)
            def _(col):
                embv[row, pl.ds(col, 16)] = one_m_g * embv[row, pl.ds(col, 16)]
            return 0
        lax.fori_loop(0, rows, escale, 0)

        start_in(0, hva, tva, hsa, tsa, p0, rows, cw)

        def pair(j, _):
            i0 = 2 * j
            i1 = 2 * j + 1

            @pl.when(j > 0)
            def _():
                out_copy(i1 - 2, hvb, osb, p0, rows).wait()
            start_in(i1, hvb, tvb, hsb, tsb, p0, rows, cw)

            wait_in(i0, hva, tva, hsa, tsa, p0, rows, cw)
            compute(hva, tva, rows)
            out_copy(i0, hva, osa, p0, rows).start()

            @pl.when(j < _B * _T // 2 - 1)
            def _():
                out_copy(i0, hva, osa, p0, rows).wait()
                start_in(i0 + 2, hva, tva, hsa, tsa, p0, rows, cw)

            wait_in(i1, hvb, tvb, hsb, tsb, p0, rows, cw)
            compute(hvb, tvb, rows)
            out_copy(i1, hvb, osb, p0, rows).start()
            return 0

        lax.fori_loop(0, _B * _T // 2, pair, 0)
        out_copy(_B * _T - 2, hva, osa, p0, rows).wait()
        out_copy(_B * _T - 1, hvb, osb, p0, rows).wait()

    do_slice(wid, _PR)
    do_slice(wid + 32, _PR)

    @pl.when(wid == 0)
    def _():
        do_slice(64, 1)


def kernel(hidden_state, aspect_ratio_ids, gate, embedding, tile_embedding_weight):
    ids = jnp.zeros((_B * 8,), jnp.int32).at[::8].set(
        aspect_ratio_ids.astype(jnp.int32))
    gate16 = jnp.broadcast_to(gate, (16,))

    mesh = plsc.VectorSubcoreMesh(core_axis_name="c", subcore_axis_name="s")
    sc = pl.kernel(
        _sc_body,
        out_type=jax.ShapeDtypeStruct((_B, _T, _P, _H), jnp.float32),
        mesh=mesh,
        scratch_types=[
            pltpu.VMEM((_PR, _H), jnp.float32),
            pltpu.VMEM((_PR, _H), jnp.float32),
            pltpu.VMEM((_PR, _H), jnp.float32),
            pltpu.VMEM((1, _PR * _H), jnp.float32),
            pltpu.VMEM((1, _PR * _H), jnp.float32),
            pltpu.VMEM((16,), jnp.float32),
            pltpu.VMEM((_B * 8,), jnp.int32),
            pltpu.SemaphoreType.DMA,
            pltpu.SemaphoreType.DMA,
            pltpu.SemaphoreType.DMA,
            pltpu.SemaphoreType.DMA,
            pltpu.SemaphoreType.DMA,
            pltpu.SemaphoreType.DMA,
        ],
        compiler_params=pltpu.CompilerParams(use_tc_tiling_on_sc=True),
    )
    return sc(hidden_state, ids, gate16, embedding, tile_embedding_weight)
